# baseline (speedup 1.0000x reference)
import functools

import jax
import jax.numpy as jnp
from jax import lax
from jax.experimental import pallas as pl
from jax.experimental.pallas import tpu as pltpu
from jax.experimental.pallas import tpu_sc as plsc

_NC = 2
_NS = 16
_NW = _NC * _NS
_R = 16


def _sc_body(seq, pe, out, b0, b1, b2, b3, pbuf, sin, sout, spe,
             *, batch, seq_len, d):
    bufs = [b0, b1, b2, b3]
    w = lax.axis_index("c") * _NS + lax.axis_index("s")
    s_per_w = seq_len // _NW
    s0 = w * s_per_w
    nr = s_per_w // _R
    nvec = d // 16

    def in_src(r, j):
        return seq.at[pl.ds(j * seq_len + s0 + r * _R, _R)]

    def out_dst(r, j):
        return out.at[pl.ds(j * seq_len + s0 + r * _R, _R)]

    def pe_src(r):
        return pe.at[pl.ds(s0 + r * _R, _R)]

    pltpu.async_copy(pe_src(0), pbuf.at[0], spe.at[0])
    pltpu.async_copy(in_src(0, 0), bufs[0], sin.at[0])

    def round_body(r, carry):
        cur = lax.rem(r, 2)

        for par in range(2):
            @pl.when(cur == par)
            def _():
                pltpu.make_async_copy(pe_src(r), pbuf.at[par], spe.at[par]).wait()

                @pl.when(r <= nr - 2)
                def _():
                    pltpu.async_copy(pe_src(r + 1), pbuf.at[1 - par],
                                     spe.at[1 - par])

        for j in range(batch):
            pltpu.make_async_copy(in_src(r, j), bufs[j], sin.at[j]).wait()

            if j + 1 < batch:
                @pl.when(r >= 1)
                def _():
                    pltpu.make_async_copy(
                        bufs[j + 1], out_dst(0, j + 1), sout.at[j + 1]).wait()
                pltpu.async_copy(in_src(r, j + 1), bufs[j + 1], sin.at[j + 1])
            else:
                @pl.when(r <= nr - 2)
                def _():
                    pltpu.make_async_copy(
                        bufs[0], out_dst(0, 0), sout.at[0]).wait()
                    pltpu.async_copy(in_src(r + 1, 0), bufs[0], sin.at[0])

            def add_row(rr, c2):
                for c in range(nvec):
                    sl = pl.ds(c * 16, 16)
                    bufs[j][rr, sl] = bufs[j][rr, sl] + pbuf[cur, rr, sl]
                return c2

            lax.fori_loop(0, _R, add_row, 0)
            pltpu.async_copy(bufs[j], out_dst(r, j), sout.at[j])
        return carry

    lax.fori_loop(0, nr, round_body, 0)

    for j in range(batch):
        pltpu.make_async_copy(bufs[j], out_dst(0, j), sout.at[j]).wait()


def kernel(sequence, pe):
    b, s, d = sequence.shape
    rows = b * s
    seq2d = sequence.reshape(rows, d)
    mesh = plsc.VectorSubcoreMesh(
        core_axis_name="c", subcore_axis_name="s",
        num_cores=_NC, num_subcores=_NS,
    )
    body = functools.partial(_sc_body, batch=b, seq_len=s, d=d)
    out2d = pl.kernel(
        body,
        out_type=jax.ShapeDtypeStruct((rows, d), sequence.dtype),
        mesh=mesh,
        scratch_types=[
            pltpu.VMEM((_R, d), sequence.dtype),
            pltpu.VMEM((_R, d), sequence.dtype),
            pltpu.VMEM((_R, d), sequence.dtype),
            pltpu.VMEM((_R, d), sequence.dtype),
            pltpu.VMEM((2, _R, d), sequence.dtype),
            pltpu.SemaphoreType.DMA((4,)),
            pltpu.SemaphoreType.DMA((4,)),
            pltpu.SemaphoreType.DMA((2,)),
        ],
    )(seq2d, pe)
    return out2d.reshape(b, s, d)


# speedup vs baseline: 1.1490x; 1.1490x over previous
import functools

import jax
import jax.numpy as jnp
from jax import lax
from jax.experimental import pallas as pl
from jax.experimental.pallas import tpu as pltpu
from jax.experimental.pallas import tpu_sc as plsc

_BS = 2048

_NC = 2
_NS = 16
_NW = _NC * _NS
_R = 32


def _tc_add_kernel(seq_ref, pe_ref, out_ref):
    out_ref[...] = seq_ref[...] + pe_ref[...]


def _tc_part(seq, pe):
    b, s, d = seq.shape
    ns = s // _BS
    return pl.pallas_call(
        _tc_add_kernel,
        grid=(ns, b),
        in_specs=[
            pl.BlockSpec((1, _BS, d), lambda i, j: (j, i, 0)),
            pl.BlockSpec((_BS, d), lambda i, j: (i, 0)),
        ],
        out_specs=pl.BlockSpec((1, _BS, d), lambda i, j: (j, i, 0)),
        out_shape=jax.ShapeDtypeStruct((b, s, d), seq.dtype),
    )(seq, pe)


def _sc_body(seq, pe, out, buf, pbuf, *, seq_len, d):
    w = lax.axis_index("c") * _NS + lax.axis_index("s")
    s_per_w = seq_len // _NW
    s0 = w * s_per_w
    nvec = d // 16

    def chunk(i, carry):
        base = s0 + i * _R
        pltpu.sync_copy(pe.at[pl.ds(base, _R)], pbuf)
        pltpu.sync_copy(seq.at[pl.ds(base, _R)], buf)

        def add_row(r, c2):
            for c in range(nvec):
                sl = pl.ds(c * 16, 16)
                buf[r, sl] = buf[r, sl] + pbuf[r, sl]
            return c2

        lax.fori_loop(0, _R, add_row, 0)
        pltpu.sync_copy(buf, out.at[pl.ds(base, _R)])
        return carry

    lax.fori_loop(0, s_per_w // _R, chunk, 0)


def _sc_part(seq2d, pe):
    s, d = seq2d.shape
    mesh = plsc.VectorSubcoreMesh(
        core_axis_name="c", subcore_axis_name="s",
        num_cores=_NC, num_subcores=_NS,
    )
    body = functools.partial(_sc_body, seq_len=s, d=d)
    return pl.kernel(
        body,
        out_type=jax.ShapeDtypeStruct((s, d), seq2d.dtype),
        mesh=mesh,
        scratch_types=[
            pltpu.VMEM((_R, d), seq2d.dtype),
            pltpu.VMEM((_R, d), seq2d.dtype),
        ],
    )(seq2d, pe)


def kernel(sequence, pe):
    b, s, d = sequence.shape
    out_tc = _tc_part(sequence[: b - 1], pe)
    out_sc = _sc_part(sequence[b - 1], pe)
    return jnp.concatenate([out_tc, out_sc[None]], axis=0)


# speedup vs baseline: 3.7755x; 3.2859x over previous
import jax
import jax.numpy as jnp
from jax.experimental import pallas as pl

_BS = 2048


def _add_kernel(seq_ref, pe_ref, out_ref):
    out_ref[...] = seq_ref[...] + pe_ref[...]


def kernel(sequence, pe):
    b, s, d = sequence.shape
    ns = s // _BS
    return pl.pallas_call(
        _add_kernel,
        grid=(ns, b),
        in_specs=[
            pl.BlockSpec((1, _BS, d), lambda i, j: (j, i, 0)),
            pl.BlockSpec((_BS, d), lambda i, j: (i, 0)),
        ],
        out_specs=pl.BlockSpec((1, _BS, d), lambda i, j: (j, i, 0)),
        out_shape=jax.ShapeDtypeStruct((b, s, d), sequence.dtype),
    )(sequence, pe)
